# baseline (speedup 1.0000x reference)
import functools

import jax
import jax.numpy as jnp
from jax import lax
from jax.experimental import pallas as pl
from jax.experimental.pallas import tpu as pltpu
from jax.experimental.pallas import tpu_sc as plsc

B = 4
S = 2048
H = 128
TAB = 500000
NSLOT = 7

_CMOD = (1, 256, 65536, 277216, 467296, 127776, 210656, 427936)
_WRAP = 448384

_NC = 2
_NS = 16
_NW = _NC * _NS

_CHUNK = 128
_CPB = S // _CHUNK
_TPW = (B * _CPB) // _NW
_NTASK = NSLOT * _TPW
_NBUF = 6


def _sc_gather_body(x_hbm, byte_hbm, t3, t4, t5, t6, t7, t8, out_hbm,
                    xv, idxv, rowsv0, rowsv1, rowsv2, rowsv3, rowsv4, rowsv5,
                    gsem0, gsem1, gsem2, gsem3, gsem4, gsem5,
                    wsem0, wsem1, wsem2, wsem3, wsem4, wsem5):
    wid = lax.axis_index("s") * _NC + lax.axis_index("c")
    pltpu.sync_copy(x_hbm, xv.at[pl.ds(0, B * S)])
    xv[pl.ds(B * S, 16)] = jnp.zeros((16,), jnp.int32)

    tables = (t3, t4, t5, t6, t7, t8, byte_hbm)
    c500k = jnp.full((16,), 500000, jnp.int32)
    zeros16 = jnp.zeros((16,), jnp.int32)
    wrap16 = jnp.full((16,), _WRAP, jnp.int32)

    rowsv = (rowsv0, rowsv1, rowsv2, rowsv3, rowsv4, rowsv5)
    gsem = (gsem0, gsem1, gsem2, gsem3, gsem4, gsem5)
    wsem = (wsem0, wsem1, wsem2, wsem3, wsem4, wsem5)

    def compute_idx_all(c, base):
        for g in range(_CHUNK // 16):
            off = base + g * 16
            acc = xv[pl.ds(off, 16)]
            idxv[c * NSLOT + 6, pl.ds(g * 16, 16)] = acc
            for ii in (1, 2):
                acc = acc + xv[pl.ds(off + ii, 16)] * _CMOD[ii]
            for slot in range(6):
                n = slot + 3
                if n > 3:
                    acc = acc + xv[pl.ds(off + n - 1, 16)] * _CMOD[n - 1]
                h = acc
                if n == 8:
                    x7 = xv[pl.ds(off + 7, 16)]
                    h = h + jnp.where(x7 >= 128, wrap16, zeros16)
                h = jnp.maximum(lax.rem(h, c500k), zeros16)
                idxv[c * NSLOT + slot, pl.ds(g * 16, 16)] = h

    tasks = [(c, slot) for c in range(_TPW) for slot in range(NSLOT)]
    pend_g = [None] * _NTASK
    pend_w = [None] * _NBUF
    out_base_of = [None] * _NTASK
    for i, (c, slot) in enumerate(tasks):
        p = i % _NBUF
        if pend_w[p] is not None:
            pend_w[p].wait()
            pend_w[p] = None
        chunk = wid * _TPW + c
        b = chunk // _CPB
        t0 = (chunk - b * _CPB) * _CHUNK
        if slot == 0:
            compute_idx_all(c, b * S + t0)
        out_base_of[i] = (slot * B + b) * S + t0
        pend_g[i] = pltpu.async_copy(
            tables[slot].at[idxv.at[jnp.int32(c * NSLOT + slot)]], rowsv[p],
            gsem[p])
        m = i - (_NBUF - 1)
        if m >= 0:
            q = m % _NBUF
            pend_g[m].wait()
            pend_w[q] = pltpu.async_copy(
                rowsv[q], out_hbm.at[pl.ds(out_base_of[m], _CHUNK)], wsem[q])
    for m in range(max(0, _NTASK - _NBUF + 1), _NTASK):
        q = m % _NBUF
        pend_g[m].wait()
        pend_w[q] = pltpu.async_copy(
            rowsv[q], out_hbm.at[pl.ds(out_base_of[m], _CHUNK)], wsem[q])
    for q in range(_NBUF):
        if pend_w[q] is not None:
            pend_w[q].wait()


@functools.cache
def _build_sc_gather():
    mesh = plsc.VectorSubcoreMesh(core_axis_name="c", subcore_axis_name="s")
    return pl.kernel(
        _sc_gather_body,
        out_type=jax.ShapeDtypeStruct((NSLOT * B * S, H), jnp.float32),
        mesh=mesh,
        scratch_types=(
            [pltpu.VMEM((B * S + 16,), jnp.int32),
             pltpu.VMEM((_NTASK, _CHUNK), jnp.int32)]
            + [pltpu.VMEM((_CHUNK, H), jnp.float32) for _ in range(_NBUF)]
            + [pltpu.SemaphoreType.DMA for _ in range(2 * _NBUF)]
        ),
    )


_TBLK = 512


def _tc_body(f_ref, w_ref, b_ref, o_ref):
    tb = pl.program_id(1)
    acc = f_ref[6, 0] + b_ref[0][None, :]
    row = lax.broadcasted_iota(jnp.int32, (_TBLK, H), 0) + tb * _TBLK
    for k in range(6):
        n = k + 3
        f = f_ref[k, 0]
        f = jnp.where(row <= S - n, f, 0.0)
        wk = w_ref[:, k * H:(k + 1) * H]
        acc = acc + lax.dot_general(
            f, wk, (((1,), (1,)), ((), ())),
            preferred_element_type=jnp.float32)
    o_ref[0] = acc


_tc_project = pl.pallas_call(
    _tc_body,
    grid=(B, S // _TBLK),
    in_specs=[
        pl.BlockSpec((NSLOT, 1, _TBLK, H),
                     lambda b, t: (jnp.int32(0), b, t, jnp.int32(0))),
        pl.BlockSpec((H, 6 * H), lambda b, t: (jnp.int32(0), jnp.int32(0))),
        pl.BlockSpec((1, H), lambda b, t: (jnp.int32(0), jnp.int32(0))),
    ],
    out_specs=pl.BlockSpec((1, _TBLK, H), lambda b, t: (b, t, jnp.int32(0))),
    out_shape=jax.ShapeDtypeStruct((B, S, H), jnp.float32),
)


def kernel(x, byte_table, ngram_3, ngram_4, ngram_5, ngram_6, ngram_7,
           ngram_8, W, b):
    x32 = x.astype(jnp.int32).reshape(B * S)
    feats = _build_sc_gather()(x32, byte_table, ngram_3, ngram_4, ngram_5,
                               ngram_6, ngram_7, ngram_8)
    feats = feats.reshape(NSLOT, B, S, H)
    return _tc_project(feats, W, b.reshape(1, H))


# speedup vs baseline: 1.1334x; 1.1334x over previous
import functools

import jax
import jax.numpy as jnp
from jax import lax
from jax.experimental import pallas as pl
from jax.experimental.pallas import tpu as pltpu
from jax.experimental.pallas import tpu_sc as plsc

B = 4
S = 2048
H = 128
TAB = 500000
NSLOT = 7

_CMOD = (1, 256, 65536, 277216, 467296, 127776, 210656, 427936)
_WRAP = 448384

_NC = 2
_NS = 16
_NW = _NC * _NS

_CHUNK = 128
_CPB = S // _CHUNK
_TPW = (B * _CPB) // _NW
_NTASK = NSLOT * _TPW
_NBUF = 6


def _sc_gather_body(x_hbm, byte_hbm, t3, t4, t5, t6, t7, t8, out_hbm,
                    xv, idxv, rowsv0, rowsv1, rowsv2, rowsv3, rowsv4, rowsv5,
                    gsem0, gsem1, gsem2, gsem3, gsem4, gsem5,
                    wsem0, wsem1, wsem2, wsem3, wsem4, wsem5):
    wid = lax.axis_index("s") * _NC + lax.axis_index("c")
    pltpu.sync_copy(x_hbm, xv.at[pl.ds(0, B * S)])
    xv[pl.ds(B * S, 16)] = jnp.zeros((16,), jnp.int32)

    tables = (t3, t4, t5, t6, t7, t8, byte_hbm)
    c500k = jnp.full((16,), 500000, jnp.int32)
    zeros16 = jnp.zeros((16,), jnp.int32)
    wrap16 = jnp.full((16,), _WRAP, jnp.int32)
    inv500k = jnp.full((16,), 1.0 / 500000.0, jnp.float32)

    def mod500k(v):
        v = jnp.maximum(v, zeros16)
        q = (v.astype(jnp.float32) * inv500k).astype(jnp.int32)
        r = v - q * 500000
        r = jnp.where(r < 0, r + c500k, r)
        r = jnp.where(r >= 500000, r - c500k, r)
        return r

    rowsv = (rowsv0, rowsv1, rowsv2, rowsv3, rowsv4, rowsv5)
    gsem = (gsem0, gsem1, gsem2, gsem3, gsem4, gsem5)
    wsem = (wsem0, wsem1, wsem2, wsem3, wsem4, wsem5)

    def compute_idx_all(c, base):
        for g in range(_CHUNK // 16):
            off = base + g * 16
            acc = xv[pl.ds(off, 16)]
            idxv[c * NSLOT + 6, pl.ds(g * 16, 16)] = acc
            for ii in (1, 2):
                acc = acc + xv[pl.ds(off + ii, 16)] * _CMOD[ii]
            for slot in range(6):
                n = slot + 3
                if n > 3:
                    acc = acc + xv[pl.ds(off + n - 1, 16)] * _CMOD[n - 1]
                h = acc
                if n == 8:
                    x7 = xv[pl.ds(off + 7, 16)]
                    h = h + jnp.where(x7 >= 128, wrap16, zeros16)
                idxv[c * NSLOT + slot, pl.ds(g * 16, 16)] = mod500k(h)

    tasks = [(c, slot) for c in range(_TPW) for slot in range(NSLOT)]
    pend_g = [None] * _NTASK
    pend_w = [None] * _NBUF
    out_base_of = [None] * _NTASK
    for i, (c, slot) in enumerate(tasks):
        p = i % _NBUF
        if pend_w[p] is not None:
            pend_w[p].wait()
            pend_w[p] = None
        chunk = wid * _TPW + c
        b = chunk // _CPB
        t0 = (chunk - b * _CPB) * _CHUNK
        if slot == 0:
            compute_idx_all(c, b * S + t0)
        out_base_of[i] = (slot * B + b) * S + t0
        pend_g[i] = pltpu.async_copy(
            tables[slot].at[idxv.at[jnp.int32(c * NSLOT + slot)]], rowsv[p],
            gsem[p])
        m = i - (_NBUF - 1)
        if m >= 0:
            q = m % _NBUF
            pend_g[m].wait()
            pend_w[q] = pltpu.async_copy(
                rowsv[q], out_hbm.at[pl.ds(out_base_of[m], _CHUNK)], wsem[q])
    for m in range(max(0, _NTASK - _NBUF + 1), _NTASK):
        q = m % _NBUF
        pend_g[m].wait()
        pend_w[q] = pltpu.async_copy(
            rowsv[q], out_hbm.at[pl.ds(out_base_of[m], _CHUNK)], wsem[q])
    for q in range(_NBUF):
        if pend_w[q] is not None:
            pend_w[q].wait()


@functools.cache
def _build_sc_gather():
    mesh = plsc.VectorSubcoreMesh(core_axis_name="c", subcore_axis_name="s")
    return pl.kernel(
        _sc_gather_body,
        out_type=jax.ShapeDtypeStruct((NSLOT * B * S, H), jnp.float32),
        mesh=mesh,
        scratch_types=(
            [pltpu.VMEM((B * S + 16,), jnp.int32),
             pltpu.VMEM((_NTASK, _CHUNK), jnp.int32)]
            + [pltpu.VMEM((_CHUNK, H), jnp.float32) for _ in range(_NBUF)]
            + [pltpu.SemaphoreType.DMA for _ in range(2 * _NBUF)]
        ),
    )


_TBLK = 512


def _tc_body(f_ref, w_ref, b_ref, o_ref):
    tb = pl.program_id(1)
    acc = f_ref[6, 0] + b_ref[0][None, :]
    row = lax.broadcasted_iota(jnp.int32, (_TBLK, H), 0) + tb * _TBLK
    for k in range(6):
        n = k + 3
        f = f_ref[k, 0]
        f = jnp.where(row <= S - n, f, 0.0)
        wk = w_ref[:, k * H:(k + 1) * H]
        acc = acc + lax.dot_general(
            f, wk, (((1,), (1,)), ((), ())),
            preferred_element_type=jnp.float32)
    o_ref[0] = acc


_tc_project = pl.pallas_call(
    _tc_body,
    grid=(B, S // _TBLK),
    in_specs=[
        pl.BlockSpec((NSLOT, 1, _TBLK, H),
                     lambda b, t: (jnp.int32(0), b, t, jnp.int32(0))),
        pl.BlockSpec((H, 6 * H), lambda b, t: (jnp.int32(0), jnp.int32(0))),
        pl.BlockSpec((1, H), lambda b, t: (jnp.int32(0), jnp.int32(0))),
    ],
    out_specs=pl.BlockSpec((1, _TBLK, H), lambda b, t: (b, t, jnp.int32(0))),
    out_shape=jax.ShapeDtypeStruct((B, S, H), jnp.float32),
)


def kernel(x, byte_table, ngram_3, ngram_4, ngram_5, ngram_6, ngram_7,
           ngram_8, W, b):
    x32 = x.astype(jnp.int32).reshape(B * S)
    feats = _build_sc_gather()(x32, byte_table, ngram_3, ngram_4, ngram_5,
                               ngram_6, ngram_7, ngram_8)
    feats = feats.reshape(NSLOT, B, S, H)
    return _tc_project(feats, W, b.reshape(1, H))


# speedup vs baseline: 1.2277x; 1.0832x over previous
import functools

import jax
import jax.numpy as jnp
from jax import lax
from jax.experimental import pallas as pl
from jax.experimental.pallas import tpu as pltpu
from jax.experimental.pallas import tpu_sc as plsc

B = 4
S = 2048
H = 128
TAB = 500000
NSLOT = 7

_CMOD = (1, 256, 65536, 277216, 467296, 127776, 210656, 427936)
_WRAP = 448384

_NC = 2
_NS = 16
_NW = _NC * _NS

_CHUNK = 128
_CPB = S // _CHUNK
_TPW = (B * _CPB) // _NW
_NTASK = NSLOT * _TPW
_NBUF = 6


def _sc_gather_body(x_hbm, byte_hbm, t3, t4, t5, t6, t7, t8, out_hbm,
                    xv, idxv, rowsv0, rowsv1, rowsv2, rowsv3, rowsv4, rowsv5,
                    gsem0, gsem1, gsem2, gsem3, gsem4, gsem5,
                    wsem0, wsem1, wsem2, wsem3, wsem4, wsem5):
    wid = lax.axis_index("s") * _NC + lax.axis_index("c")
    pltpu.sync_copy(x_hbm, xv.at[pl.ds(0, B * S)])
    xv[pl.ds(B * S, 16)] = jnp.zeros((16,), jnp.int32)

    tables = (t3, t4, t5, t6, t7, t8, byte_hbm)
    c500k = jnp.full((16,), 500000, jnp.int32)
    zeros16 = jnp.zeros((16,), jnp.int32)
    wrap16 = jnp.full((16,), _WRAP, jnp.int32)
    inv500k = jnp.full((16,), 1.0 / 500000.0, jnp.float32)

    def mod500k(v):
        v = jnp.maximum(v, zeros16)
        q = (v.astype(jnp.float32) * inv500k).astype(jnp.int32)
        r = v - q * 500000
        r = jnp.where(r < 0, r + c500k, r)
        r = jnp.where(r >= 500000, r - c500k, r)
        return r

    rowsv = (rowsv0, rowsv1, rowsv2, rowsv3, rowsv4, rowsv5)
    gsem = (gsem0, gsem1, gsem2, gsem3, gsem4, gsem5)
    wsem = (wsem0, wsem1, wsem2, wsem3, wsem4, wsem5)

    def compute_idx_all(c, base):
        for g in range(_CHUNK // 16):
            off = base + g * 16
            acc = xv[pl.ds(off, 16)]
            idxv[c * NSLOT + 6, pl.ds(g * 16, 16)] = acc
            for ii in (1, 2):
                acc = acc + xv[pl.ds(off + ii, 16)] * _CMOD[ii]
            for slot in range(6):
                n = slot + 3
                if n > 3:
                    acc = acc + xv[pl.ds(off + n - 1, 16)] * _CMOD[n - 1]
                h = acc
                if n == 8:
                    x7 = xv[pl.ds(off + 7, 16)]
                    h = h + jnp.where(x7 >= 128, wrap16, zeros16)
                idxv[c * NSLOT + slot, pl.ds(g * 16, 16)] = mod500k(h)

    tasks = [(c, slot) for c in range(_TPW) for slot in range(NSLOT)]
    pend_g = [None] * _NTASK
    pend_w = [None] * _NBUF
    out_base_of = [None] * _NTASK
    for i, (c, slot) in enumerate(tasks):
        p = i % _NBUF
        if pend_w[p] is not None:
            pend_w[p].wait()
            pend_w[p] = None
        chunk = wid * _TPW + c
        b = chunk // _CPB
        t0 = (chunk - b * _CPB) * _CHUNK
        if slot == 0:
            compute_idx_all(c, b * S + t0)
        out_base_of[i] = (slot * B + b) * S + t0
        pend_g[i] = pltpu.async_copy(
            tables[slot].at[idxv.at[jnp.int32(c * NSLOT + slot)]], rowsv[p],
            gsem[p])
        m = i - (_NBUF - 1)
        if m >= 0:
            q = m % _NBUF
            pend_g[m].wait()
            pend_w[q] = pltpu.async_copy(
                rowsv[q], out_hbm.at[pl.ds(out_base_of[m], _CHUNK)], wsem[q])
    for m in range(max(0, _NTASK - _NBUF + 1), _NTASK):
        q = m % _NBUF
        pend_g[m].wait()
        pend_w[q] = pltpu.async_copy(
            rowsv[q], out_hbm.at[pl.ds(out_base_of[m], _CHUNK)], wsem[q])
    for q in range(_NBUF):
        if pend_w[q] is not None:
            pend_w[q].wait()


@functools.cache
def _build_sc_gather():
    mesh = plsc.VectorSubcoreMesh(core_axis_name="c", subcore_axis_name="s")
    return pl.kernel(
        _sc_gather_body,
        out_type=jax.ShapeDtypeStruct((NSLOT * B * S, H), jnp.float32),
        mesh=mesh,
        scratch_types=(
            [pltpu.VMEM((B * S + 16,), jnp.int32),
             pltpu.VMEM((_NTASK, _CHUNK), jnp.int32)]
            + [pltpu.VMEM((_CHUNK, H), jnp.float32) for _ in range(_NBUF)]
            + [pltpu.SemaphoreType.DMA for _ in range(2 * _NBUF)]
        ),
    )


_TBLK = 1024


def _tc_body(f_ref, w_ref, b_ref, o_ref):
    tb = pl.program_id(1)
    acc = f_ref[6, 0] + b_ref[0][None, :]
    row = lax.broadcasted_iota(jnp.int32, (_TBLK, H), 0) + tb * _TBLK
    for k in range(6):
        n = k + 3
        f = f_ref[k, 0]
        f = jnp.where(row <= S - n, f, 0.0)
        wk = w_ref[:, k * H:(k + 1) * H]
        acc = acc + lax.dot_general(
            f, wk, (((1,), (1,)), ((), ())),
            preferred_element_type=jnp.float32)
    o_ref[0] = acc


_tc_project = pl.pallas_call(
    _tc_body,
    grid=(B, S // _TBLK),
    in_specs=[
        pl.BlockSpec((NSLOT, 1, _TBLK, H),
                     lambda b, t: (jnp.int32(0), b, t, jnp.int32(0))),
        pl.BlockSpec((H, 6 * H), lambda b, t: (jnp.int32(0), jnp.int32(0))),
        pl.BlockSpec((1, H), lambda b, t: (jnp.int32(0), jnp.int32(0))),
    ],
    out_specs=pl.BlockSpec((1, _TBLK, H), lambda b, t: (b, t, jnp.int32(0))),
    out_shape=jax.ShapeDtypeStruct((B, S, H), jnp.float32),
)


def kernel(x, byte_table, ngram_3, ngram_4, ngram_5, ngram_6, ngram_7,
           ngram_8, W, b):
    x32 = x.astype(jnp.int32).reshape(B * S)
    feats = _build_sc_gather()(x32, byte_table, ngram_3, ngram_4, ngram_5,
                               ngram_6, ngram_7, ngram_8)
    feats = feats.reshape(NSLOT, B, S, H)
    return _tc_project(feats, W, b.reshape(1, H))


# speedup vs baseline: 1.2530x; 1.0206x over previous
import functools

import jax
import jax.numpy as jnp
from jax import lax
from jax.experimental import pallas as pl
from jax.experimental.pallas import tpu as pltpu
from jax.experimental.pallas import tpu_sc as plsc

B = 4
S = 2048
H = 128
TAB = 500000
NSLOT = 7

_CMOD = (1, 256, 65536, 277216, 467296, 127776, 210656, 427936)
_WRAP = 448384

_NC = 2
_NS = 16
_NW = _NC * _NS

_CHUNK = 128
_CPB = S // _CHUNK
_TPW = (B * _CPB) // _NW
_NTASK = NSLOT * _TPW
_NBUF = 6


def _sc_gather_body(x_hbm, byte_hbm, t3, t4, t5, t6, t7, t8, out_hbm,
                    xv, idxv, rowsv0, rowsv1, rowsv2, rowsv3, rowsv4, rowsv5,
                    gsem0, gsem1, gsem2, gsem3, gsem4, gsem5,
                    wsem0, wsem1, wsem2, wsem3, wsem4, wsem5):
    wid = lax.axis_index("s") * _NC + lax.axis_index("c")
    pltpu.sync_copy(x_hbm, xv.at[pl.ds(0, B * S)])
    xv[pl.ds(B * S, 16)] = jnp.zeros((16,), jnp.int32)

    tables = (t3, t4, t5, t6, t7, t8, byte_hbm)
    c500k = jnp.full((16,), 500000, jnp.int32)
    zeros16 = jnp.zeros((16,), jnp.int32)
    wrap16 = jnp.full((16,), _WRAP, jnp.int32)
    inv500k = jnp.full((16,), 1.0 / 500000.0, jnp.float32)

    def mod500k(v):
        v = jnp.maximum(v, zeros16)
        q = (v.astype(jnp.float32) * inv500k).astype(jnp.int32)
        r = v - q * 500000
        r = jnp.where(r < 0, r + c500k, r)
        r = jnp.where(r >= 500000, r - c500k, r)
        return r

    rowsv = (rowsv0, rowsv1, rowsv2, rowsv3, rowsv4, rowsv5)
    gsem = (gsem0, gsem1, gsem2, gsem3, gsem4, gsem5)
    wsem = (wsem0, wsem1, wsem2, wsem3, wsem4, wsem5)

    def compute_idx_all(c, base):
        for g in range(_CHUNK // 16):
            off = base + g * 16
            acc = xv[pl.ds(off, 16)]
            idxv[c * NSLOT + 6, pl.ds(g * 16, 16)] = acc
            for ii in (1, 2):
                acc = acc + xv[pl.ds(off + ii, 16)] * _CMOD[ii]
            for slot in range(6):
                n = slot + 3
                if n > 3:
                    acc = acc + xv[pl.ds(off + n - 1, 16)] * _CMOD[n - 1]
                h = acc
                if n == 8:
                    x7 = xv[pl.ds(off + 7, 16)]
                    h = h + jnp.where(x7 >= 128, wrap16, zeros16)
                idxv[c * NSLOT + slot, pl.ds(g * 16, 16)] = mod500k(h)

    tasks = [(c, slot) for c in range(_TPW) for slot in range(NSLOT)]
    pend_g = [None] * _NTASK
    pend_w = [None] * _NBUF
    out_base_of = [None] * _NTASK
    for i, (c, slot) in enumerate(tasks):
        p = i % _NBUF
        if pend_w[p] is not None:
            pend_w[p].wait()
            pend_w[p] = None
        chunk = wid * _TPW + c
        b = chunk // _CPB
        t0 = (chunk - b * _CPB) * _CHUNK
        if slot == 0:
            compute_idx_all(c, b * S + t0)
        out_base_of[i] = (slot * B + b) * S + t0
        pend_g[i] = pltpu.async_copy(
            tables[slot].at[idxv.at[jnp.int32(c * NSLOT + slot)]], rowsv[p],
            gsem[p])
        m = i - (_NBUF - 1)
        if m >= 0:
            q = m % _NBUF
            pend_g[m].wait()
            pend_w[q] = pltpu.async_copy(
                rowsv[q], out_hbm.at[pl.ds(out_base_of[m], _CHUNK)], wsem[q])
    for m in range(max(0, _NTASK - _NBUF + 1), _NTASK):
        q = m % _NBUF
        pend_g[m].wait()
        pend_w[q] = pltpu.async_copy(
            rowsv[q], out_hbm.at[pl.ds(out_base_of[m], _CHUNK)], wsem[q])
    for q in range(_NBUF):
        if pend_w[q] is not None:
            pend_w[q].wait()


@functools.cache
def _build_sc_gather():
    mesh = plsc.VectorSubcoreMesh(core_axis_name="c", subcore_axis_name="s")
    return pl.kernel(
        _sc_gather_body,
        out_type=jax.ShapeDtypeStruct((NSLOT * B * S, H), jnp.float32),
        mesh=mesh,
        scratch_types=(
            [pltpu.VMEM((B * S + 16,), jnp.int32),
             pltpu.VMEM((_NTASK, _CHUNK), jnp.int32)]
            + [pltpu.VMEM((_CHUNK, H), jnp.float32) for _ in range(_NBUF)]
            + [pltpu.SemaphoreType.DMA for _ in range(2 * _NBUF)]
        ),
    )


_TBLK = 2048


def _tc_body(f_ref, w_ref, b_ref, o_ref):
    tb = pl.program_id(1)
    acc = f_ref[6, 0] + b_ref[0][None, :]
    row = lax.broadcasted_iota(jnp.int32, (_TBLK, H), 0) + tb * _TBLK
    for k in range(6):
        n = k + 3
        f = f_ref[k, 0]
        f = jnp.where(row <= S - n, f, 0.0)
        wk = w_ref[:, k * H:(k + 1) * H]
        acc = acc + lax.dot_general(
            f, wk, (((1,), (1,)), ((), ())),
            preferred_element_type=jnp.float32)
    o_ref[0] = acc


_tc_project = pl.pallas_call(
    _tc_body,
    grid=(B, S // _TBLK),
    in_specs=[
        pl.BlockSpec((NSLOT, 1, _TBLK, H),
                     lambda b, t: (jnp.int32(0), b, t, jnp.int32(0))),
        pl.BlockSpec((H, 6 * H), lambda b, t: (jnp.int32(0), jnp.int32(0))),
        pl.BlockSpec((1, H), lambda b, t: (jnp.int32(0), jnp.int32(0))),
    ],
    out_specs=pl.BlockSpec((1, _TBLK, H), lambda b, t: (b, t, jnp.int32(0))),
    out_shape=jax.ShapeDtypeStruct((B, S, H), jnp.float32),
)


def kernel(x, byte_table, ngram_3, ngram_4, ngram_5, ngram_6, ngram_7,
           ngram_8, W, b):
    x32 = x.astype(jnp.int32).reshape(B * S)
    feats = _build_sc_gather()(x32, byte_table, ngram_3, ngram_4, ngram_5,
                               ngram_6, ngram_7, ngram_8)
    feats = feats.reshape(NSLOT, B, S, H)
    return _tc_project(feats, W, b.reshape(1, H))


# speedup vs baseline: 1.4443x; 1.1527x over previous
import functools

import jax
import jax.numpy as jnp
from jax import lax
from jax.experimental import pallas as pl
from jax.experimental.pallas import tpu as pltpu
from jax.experimental.pallas import tpu_sc as plsc

B = 4
S = 2048
H = 128
TAB = 500000
NSLOT = 6

_CMOD = (1, 256, 65536, 277216, 467296, 127776, 210656, 427936)
_WRAP = 448384

_NC = 2
_NS = 16
_NW = _NC * _NS

_CHUNK = 128
_CPB = S // _CHUNK
_TPW = (B * _CPB) // _NW
_NTASK = NSLOT * _TPW
_NBUF = 6


def _sc_gather_body(x_hbm, t3, t4, t5, t6, t7, t8, out_hbm,
                    xv, idxv, rowsv0, rowsv1, rowsv2, rowsv3, rowsv4, rowsv5,
                    gsem0, gsem1, gsem2, gsem3, gsem4, gsem5,
                    wsem0, wsem1, wsem2, wsem3, wsem4, wsem5):
    wid = lax.axis_index("s") * _NC + lax.axis_index("c")
    pltpu.sync_copy(x_hbm, xv.at[pl.ds(0, B * S)])
    xv[pl.ds(B * S, 16)] = jnp.zeros((16,), jnp.int32)

    tables = (t3, t4, t5, t6, t7, t8)
    c500k = jnp.full((16,), 500000, jnp.int32)
    zeros16 = jnp.zeros((16,), jnp.int32)
    wrap16 = jnp.full((16,), _WRAP, jnp.int32)
    inv500k = jnp.full((16,), 1.0 / 500000.0, jnp.float32)

    def mod500k(v):
        v = jnp.maximum(v, zeros16)
        q = (v.astype(jnp.float32) * inv500k).astype(jnp.int32)
        r = v - q * 500000
        r = jnp.where(r < 0, r + c500k, r)
        r = jnp.where(r >= 500000, r - c500k, r)
        return r

    rowsv = (rowsv0, rowsv1, rowsv2, rowsv3, rowsv4, rowsv5)
    gsem = (gsem0, gsem1, gsem2, gsem3, gsem4, gsem5)
    wsem = (wsem0, wsem1, wsem2, wsem3, wsem4, wsem5)

    def compute_idx_all(c, base):
        for g in range(_CHUNK // 16):
            off = base + g * 16
            acc = xv[pl.ds(off, 16)]
            for ii in (1, 2):
                acc = acc + xv[pl.ds(off + ii, 16)] * _CMOD[ii]
            for slot in range(6):
                n = slot + 3
                if n > 3:
                    acc = acc + xv[pl.ds(off + n - 1, 16)] * _CMOD[n - 1]
                h = acc
                if n == 8:
                    x7 = xv[pl.ds(off + 7, 16)]
                    h = h + jnp.where(x7 >= 128, wrap16, zeros16)
                idxv[c * NSLOT + slot, pl.ds(g * 16, 16)] = mod500k(h)

    tasks = [(c, slot) for c in range(_TPW) for slot in range(NSLOT)]
    pend_g = [None] * _NTASK
    pend_w = [None] * _NBUF
    out_base_of = [None] * _NTASK
    for i, (c, slot) in enumerate(tasks):
        p = i % _NBUF
        if pend_w[p] is not None:
            pend_w[p].wait()
            pend_w[p] = None
        chunk = wid * _TPW + c
        b = chunk // _CPB
        t0 = (chunk - b * _CPB) * _CHUNK
        if slot == 0:
            compute_idx_all(c, b * S + t0)
        out_base_of[i] = (slot * B + b) * S + t0
        pend_g[i] = pltpu.async_copy(
            tables[slot].at[idxv.at[jnp.int32(c * NSLOT + slot)]], rowsv[p],
            gsem[p])
        m = i - (_NBUF - 1)
        if m >= 0:
            q = m % _NBUF
            pend_g[m].wait()
            pend_w[q] = pltpu.async_copy(
                rowsv[q], out_hbm.at[pl.ds(out_base_of[m], _CHUNK)], wsem[q])
    for m in range(max(0, _NTASK - _NBUF + 1), _NTASK):
        q = m % _NBUF
        pend_g[m].wait()
        pend_w[q] = pltpu.async_copy(
            rowsv[q], out_hbm.at[pl.ds(out_base_of[m], _CHUNK)], wsem[q])
    for q in range(_NBUF):
        if pend_w[q] is not None:
            pend_w[q].wait()


@functools.cache
def _build_sc_gather():
    mesh = plsc.VectorSubcoreMesh(core_axis_name="c", subcore_axis_name="s")
    return pl.kernel(
        _sc_gather_body,
        out_type=jax.ShapeDtypeStruct((NSLOT * B * S, H), jnp.float32),
        mesh=mesh,
        scratch_types=(
            [pltpu.VMEM((B * S + 16,), jnp.int32),
             pltpu.VMEM((_NTASK, _CHUNK), jnp.int32)]
            + [pltpu.VMEM((_CHUNK, H), jnp.float32) for _ in range(_NBUF)]
            + [pltpu.SemaphoreType.DMA for _ in range(2 * _NBUF)]
        ),
    )


_TBLK = 2048


def _tc_body(f_ref, x_ref, bt_ref, w_ref, b_ref, o_ref):
    tb = pl.program_id(1)
    xb = x_ref[0, 0]
    onehot = (jnp.reshape(xb, (_TBLK, 1))
              == lax.broadcasted_iota(jnp.int32, (_TBLK, 256), 1)
              ).astype(jnp.float32)
    acc = lax.dot_general(
        onehot, bt_ref[...], (((1,), (0,)), ((), ())),
        preferred_element_type=jnp.float32) + b_ref[0][None, :]
    row = lax.broadcasted_iota(jnp.int32, (_TBLK, H), 0) + tb * _TBLK
    for k in range(6):
        n = k + 3
        f = f_ref[k, 0]
        f = jnp.where(row <= S - n, f, 0.0)
        wk = w_ref[:, k * H:(k + 1) * H]
        acc = acc + lax.dot_general(
            f, wk, (((1,), (1,)), ((), ())),
            preferred_element_type=jnp.float32)
    o_ref[0] = acc


_tc_project = pl.pallas_call(
    _tc_body,
    grid=(B, S // _TBLK),
    in_specs=[
        pl.BlockSpec((NSLOT, 1, _TBLK, H),
                     lambda b, t: (jnp.int32(0), b, t, jnp.int32(0))),
        pl.BlockSpec((1, 1, _TBLK),
                     lambda b, t: (b, jnp.int32(0), t)),
        pl.BlockSpec((256, H), lambda b, t: (jnp.int32(0), jnp.int32(0))),
        pl.BlockSpec((H, 6 * H), lambda b, t: (jnp.int32(0), jnp.int32(0))),
        pl.BlockSpec((1, H), lambda b, t: (jnp.int32(0), jnp.int32(0))),
    ],
    out_specs=pl.BlockSpec((1, _TBLK, H), lambda b, t: (b, t, jnp.int32(0))),
    out_shape=jax.ShapeDtypeStruct((B, S, H), jnp.float32),
)


def kernel(x, byte_table, ngram_3, ngram_4, ngram_5, ngram_6, ngram_7,
           ngram_8, W, b):
    x32 = x.astype(jnp.int32)
    feats = _build_sc_gather()(x32.reshape(B * S), ngram_3, ngram_4, ngram_5,
                               ngram_6, ngram_7, ngram_8)
    feats = feats.reshape(NSLOT, B, S, H)
    return _tc_project(feats, x32.reshape(B, 1, S), byte_table, W,
                       b.reshape(1, H))
